# baseline (speedup 1.0000x reference)
import functools

import jax
import jax.numpy as jnp
from jax import lax
from jax.experimental import pallas as pl
from jax.experimental.pallas import tpu as pltpu
from jax.experimental.pallas import tpu_sc as plsc

f32 = jnp.float32
i32 = jnp.int32

N_NODES = 10000
R = 10240
JUNK = N_NODES
N_EDGES = 160000
NSUB = 16
EPS = R
BE = 128
NB = EPS // BE
STRIPE = R // NSUB

_mesh = plsc.VectorSubcoreMesh(core_axis_name="c", subcore_axis_name="s")


def _fill(ref, val, nrow, ncol):
    def row(r, carry):
        for j in range(ncol // 16):
            ref[r, pl.ds(j * 16, 16)] = jnp.full((16,), val, f32)
        return carry
    lax.fori_loop(0, nrow, row, 0)


def _deg_body(sd3, deg_hbm, idxv, onesv, table):
    cid = lax.axis_index("c")
    sid = lax.axis_index("s")
    base = sid * STRIPE
    _fill(onesv, 0.0, BE, 128)
    for k in range(STRIPE // BE):
        pltpu.sync_copy(onesv, table.at[pl.ds(base + k * BE, BE)])
    _fill(onesv, 1.0, BE, 128)
    pltpu.sync_copy(sd3.at[cid, sid], idxv)
    plsc.subcore_barrier()

    def batch(b, carry):
        pltpu.sync_copy(onesv, table.at[idxv.at[b]], add=True)
        return carry
    lax.fori_loop(0, NB, batch, 0)
    plsc.subcore_barrier()
    pltpu.sync_copy(table.at[pl.ds(base, STRIPE)],
                    deg_hbm.at[cid, pl.ds(base, STRIPE)])


def _deg_call(sd3):
    return pl.kernel(
        _deg_body,
        out_type=jax.ShapeDtypeStruct((2, R, 128), f32),
        mesh=_mesh,
        scratch_types=[
            pltpu.VMEM((NB, BE), i32),
            pltpu.VMEM((BE, 128), f32),
            pltpu.VMEM_SHARED((R, 128), f32),
        ],
    )(sd3)


NB2 = NB // 2


def _scatter_body(nchunk, dc, fstack, src3h, dst3h, out, sidxv, didxv, rowsa,
                  rowsb, table, sema, semb):
    cid = lax.axis_index("c")
    sid = lax.axis_index("s")
    base = sid * STRIPE
    c0 = nchunk // 2

    def chunk_body(chunk, oslot, halves):
        _fill(rowsa, 0.0, BE, dc)
        for k in range(STRIPE // BE):
            pltpu.sync_copy(rowsa, table.at[pl.ds(base + k * BE, BE)])
        plsc.subcore_barrier()
        src = fstack.at[chunk]

        def swait(buf, sem):
            pltpu.make_async_copy(src.at[sidxv.at[0]], buf, sem).wait()

        for h in halves:
            pltpu.sync_copy(src3h.at[sid, h], sidxv)
            pltpu.sync_copy(dst3h.at[sid, h], didxv)
            pltpu.async_copy(src.at[sidxv.at[0]], rowsa, sema)

            def pair(p, carry):
                b0 = 2 * p
                swait(rowsa, sema)
                pltpu.async_copy(src.at[sidxv.at[b0 + 1]], rowsb, semb)
                pltpu.async_copy(rowsa, table.at[didxv.at[b0]], sema,
                                 add=True)
                swait(rowsb, semb)
                swait(rowsa, sema)

                @pl.when(p < NB2 // 2 - 1)
                def _():
                    pltpu.async_copy(src.at[sidxv.at[b0 + 2]], rowsa, sema)
                pltpu.async_copy(rowsb, table.at[didxv.at[b0 + 1]], semb,
                                 add=True)
                swait(rowsb, semb)
                return carry
            lax.fori_loop(0, NB2 // 2, pair, 0)
        plsc.subcore_barrier()
        pltpu.sync_copy(table.at[pl.ds(base, STRIPE)],
                        out.at[oslot, pl.ds(base, STRIPE)])

    for i in range(c0):
        chunk_body(cid * c0 + i, cid * c0 + i, (0, 1))
    if nchunk % 2:
        chunk_body(nchunk - 1, nchunk - 1 + cid, (cid,))


def _scatter_call(fstack, src3h, dst3h):
    nchunk, _, dc = fstack.shape
    body = functools.partial(_scatter_body, nchunk, dc)
    return pl.kernel(
        body,
        out_type=jax.ShapeDtypeStruct((nchunk + nchunk % 2, R, dc), f32),
        mesh=_mesh,
        scratch_types=[
            pltpu.VMEM((NB2, BE), i32),
            pltpu.VMEM((NB2, BE), i32),
            pltpu.VMEM((BE, dc), f32),
            pltpu.VMEM((BE, dc), f32),
            pltpu.VMEM_SHARED((R, dc), f32),
            pltpu.SemaphoreType.DMA,
            pltpu.SemaphoreType.DMA,
        ],
    )(fstack, src3h, dst3h)


def _m1_body(p_ref, w1_ref, b1_ref, ni_ref, no_ref, out_ref):
    acc = jnp.dot(p_ref[:, :], w1_ref[:, :], preferred_element_type=f32)
    out_ref[:, :] = jnp.maximum(acc * ni_ref[:, :] + b1_ref[:, :], 0.0) * no_ref[:, :]


def _m1_call(P, W1, b1, ni, no):
    bm = 1024
    hd = W1.shape[1]
    return pl.pallas_call(
        _m1_body,
        grid=(R // bm,),
        in_specs=[
            pl.BlockSpec((bm, P.shape[1]), lambda i: (i, 0)),
            pl.BlockSpec((P.shape[1], hd), lambda i: (0, 0)),
            pl.BlockSpec((1, hd), lambda i: (0, 0)),
            pl.BlockSpec((bm, 1), lambda i: (i, 0)),
            pl.BlockSpec((bm, 1), lambda i: (i, 0)),
        ],
        out_specs=pl.BlockSpec((bm, hd), lambda i: (i, 0)),
        out_shape=jax.ShapeDtypeStruct((R, hd), f32),
    )(P, W1, b1.reshape(1, hd), ni.reshape(R, 1), no.reshape(R, 1))


def _m2_body(x_ref, w2_ref, b2_ref, xw_ref, xb2_ref):
    xw_ref[:, :] = lax.dot_general(
        x_ref[:, :], w2_ref[:, :], (((1,), (1,)), ((), ())),
        preferred_element_type=f32)
    xb2_ref[:, :] = jnp.dot(x_ref[:, :], b2_ref[:, :],
                            preferred_element_type=f32)


def _m2_call(X, W2, b2):
    bm = 1024
    bt, od = X.shape
    hd = W2.shape[0]
    return pl.pallas_call(
        _m2_body,
        grid=(bt // bm,),
        in_specs=[
            pl.BlockSpec((bm, od), lambda i: (i, 0)),
            pl.BlockSpec((hd, od), lambda i: (0, 0)),
            pl.BlockSpec((od, 1), lambda i: (0, 0)),
        ],
        out_specs=[
            pl.BlockSpec((bm, hd), lambda i: (i, 0)),
            pl.BlockSpec((bm, 1), lambda i: (i, 0)),
        ],
        out_shape=[
            jax.ShapeDtypeStruct((bt, hd), f32),
            jax.ShapeDtypeStruct((bt, 1), f32),
        ],
    )(X, W2, b2.reshape(od, 1))


def _m3_body(xw_ref, q_ref, ni_ref, xb2_ref, out_ref):
    s = lax.dot_general(xw_ref[:, :], q_ref[:, :], (((1,), (1,)), ((), ())),
                        preferred_element_type=f32)
    out_ref[:, :] = jax.nn.sigmoid(s * ni_ref[:, :] + xb2_ref[:, :])


def _m3_call(XW, Qp, ni_row, xb2):
    bm, bn = 1024, 1280
    bt, hd = XW.shape
    return pl.pallas_call(
        _m3_body,
        grid=(bt // bm, R // bn),
        in_specs=[
            pl.BlockSpec((bm, hd), lambda i, j: (i, 0)),
            pl.BlockSpec((bn, hd), lambda i, j: (j, 0)),
            pl.BlockSpec((1, bn), lambda i, j: (0, j)),
            pl.BlockSpec((bm, 1), lambda i, j: (i, 0)),
        ],
        out_specs=pl.BlockSpec((bm, bn), lambda i, j: (i, j)),
        out_shape=jax.ShapeDtypeStruct((bt, N_NODES), f32),
    )(XW, Qp, ni_row, xb2)


def kernel(X, label_embeds, edge_index, W1, b1, W2, b2):
    ei = edge_index.astype(i32)
    epr = N_EDGES // NSUB
    pad = jnp.full((NSUB, EPS - epr), JUNK, i32)
    src3 = jnp.concatenate([ei[0].reshape(NSUB, epr), pad], 1).reshape(NSUB, NB, BE)
    dst3 = jnp.concatenate([ei[1].reshape(NSUB, epr), pad], 1).reshape(NSUB, NB, BE)
    sd3 = jnp.stack([src3, dst3])
    src3h = src3.reshape(NSUB, 2, NB2, BE)
    dst3h = dst3.reshape(NSUB, 2, NB2, BE)

    deg = _deg_call(sd3)
    norm_out = lax.rsqrt(jnp.maximum(deg[0, :, 0], 1.0))
    norm_in = lax.rsqrt(jnp.maximum(deg[1, :, 0], 1.0))

    in_dim = label_embeds.shape[1]
    F0 = jnp.concatenate(
        [label_embeds, jnp.zeros((R - N_NODES, in_dim), f32)], 0
    ) * norm_out[:, None]
    F0s = F0.reshape(R, 2, in_dim // 2).transpose(1, 0, 2)
    P = _scatter_call(F0s, src3h, dst3h)
    P = P.transpose(1, 0, 2).reshape(R, in_dim)

    F1 = _m1_call(P, W1, b1, norm_in, norm_out)
    hd = F1.shape[1]
    F1s = F1.reshape(R, 9, hd // 9).transpose(1, 0, 2)
    Q = _scatter_call(F1s, src3h, dst3h)
    Q = jnp.concatenate([Q[:8], (Q[8] + Q[9])[None]], 0)
    Qp = Q.transpose(1, 0, 2).reshape(R, hd)

    XW, xb2 = _m2_call(X, W2, b2)
    return _m3_call(XW, Qp, norm_in.reshape(1, R), xb2)


# speedup vs baseline: 1.0337x; 1.0337x over previous
import functools

import jax
import jax.numpy as jnp
from jax import lax
from jax.experimental import pallas as pl
from jax.experimental.pallas import tpu as pltpu
from jax.experimental.pallas import tpu_sc as plsc

f32 = jnp.float32
i32 = jnp.int32

N_NODES = 10000
R = 10240
JUNK = N_NODES
N_EDGES = 160000
NSUB = 16
EPS = R
BE = 128
NB = EPS // BE
STRIPE = R // NSUB

_mesh = plsc.VectorSubcoreMesh(core_axis_name="c", subcore_axis_name="s")


def _fill(ref, val, nrow, ncol):
    def row(r, carry):
        for j in range(ncol // 16):
            ref[r, pl.ds(j * 16, 16)] = jnp.full((16,), val, f32)
        return carry
    lax.fori_loop(0, nrow, row, 0)


def _deg_body(sd3, deg_hbm, idxv, onesv, table):
    cid = lax.axis_index("c")
    sid = lax.axis_index("s")
    base = sid * STRIPE
    _fill(onesv, 0.0, BE, 128)
    for k in range(STRIPE // BE):
        pltpu.sync_copy(onesv, table.at[pl.ds(base + k * BE, BE)])
    _fill(onesv, 1.0, BE, 128)
    pltpu.sync_copy(sd3.at[cid, sid], idxv)
    plsc.subcore_barrier()

    def batch(b, carry):
        pltpu.sync_copy(onesv, table.at[idxv.at[b]], add=True)
        return carry
    lax.fori_loop(0, NB, batch, 0)
    plsc.subcore_barrier()
    pltpu.sync_copy(table.at[pl.ds(base, STRIPE)],
                    deg_hbm.at[cid, pl.ds(base, STRIPE)])


def _deg_call(sd3):
    return pl.kernel(
        _deg_body,
        out_type=jax.ShapeDtypeStruct((2, R, 128), f32),
        mesh=_mesh,
        scratch_types=[
            pltpu.VMEM((NB, BE), i32),
            pltpu.VMEM((BE, 128), f32),
            pltpu.VMEM_SHARED((R, 128), f32),
        ],
    )(sd3)


NB2 = NB // 2


def _scatter_body(nchunk, dc, fstack, src3h, dst3h, out, sidxv, didxv, rowsa,
                  rowsb, table, sema, semb):
    cid = lax.axis_index("c")
    sid = lax.axis_index("s")
    base = sid * STRIPE
    c0 = nchunk // 2

    def chunk_body(chunk, oslot, halves):
        _fill(rowsa, 0.0, BE, dc)
        for k in range(STRIPE // BE):
            pltpu.sync_copy(rowsa, table.at[pl.ds(base + k * BE, BE)])
        plsc.subcore_barrier()
        src = fstack.at[chunk]

        def swait(buf, sem):
            pltpu.make_async_copy(src.at[sidxv.at[0]], buf, sem).wait()

        for h in halves:
            pltpu.sync_copy(src3h.at[sid, h], sidxv)
            pltpu.sync_copy(dst3h.at[sid, h], didxv)
            pltpu.async_copy(src.at[sidxv.at[0]], rowsa, sema)

            def pair(p, carry):
                b0 = 2 * p
                swait(rowsa, sema)
                pltpu.async_copy(src.at[sidxv.at[b0 + 1]], rowsb, semb)
                pltpu.async_copy(rowsa, table.at[didxv.at[b0]], sema,
                                 add=True)
                swait(rowsb, semb)
                swait(rowsa, sema)

                @pl.when(p < NB2 // 2 - 1)
                def _():
                    pltpu.async_copy(src.at[sidxv.at[b0 + 2]], rowsa, sema)
                pltpu.async_copy(rowsb, table.at[didxv.at[b0 + 1]], semb,
                                 add=True)
                swait(rowsb, semb)
                return carry
            lax.fori_loop(0, NB2 // 2, pair, 0)
        plsc.subcore_barrier()
        pltpu.sync_copy(table.at[pl.ds(base, STRIPE)],
                        out.at[oslot, pl.ds(base, STRIPE)])

    for i in range(c0):
        chunk_body(cid * c0 + i, cid * c0 + i, (0, 1))
    if nchunk % 2:
        chunk_body(nchunk - 1, nchunk - 1 + cid, (cid,))


def _scatter_call(fstack, src3h, dst3h):
    nchunk, _, dc = fstack.shape
    body = functools.partial(_scatter_body, nchunk, dc)
    return pl.kernel(
        body,
        out_type=jax.ShapeDtypeStruct((nchunk + nchunk % 2, R, dc), f32),
        mesh=_mesh,
        scratch_types=[
            pltpu.VMEM((NB2, BE), i32),
            pltpu.VMEM((NB2, BE), i32),
            pltpu.VMEM((BE, dc), f32),
            pltpu.VMEM((BE, dc), f32),
            pltpu.VMEM_SHARED((R, dc), f32),
            pltpu.SemaphoreType.DMA,
            pltpu.SemaphoreType.DMA,
        ],
    )(fstack, src3h, dst3h)


def _m1_body(p_ref, w1_ref, b1_ref, ni_ref, no_ref, out_ref):
    acc = jnp.dot(p_ref[:, :], w1_ref[:, :], preferred_element_type=f32)
    out_ref[0, :, :] = (jnp.maximum(acc * ni_ref[:, :] + b1_ref[:, :], 0.0)
                        * no_ref[:, :])


def _m1_call(P, W1, b1, ni, no):
    bm = 1024
    hd = W1.shape[1]
    nc = hd // 128
    return pl.pallas_call(
        _m1_body,
        grid=(R // bm, nc),
        in_specs=[
            pl.BlockSpec((bm, P.shape[1]), lambda i, j: (i, 0)),
            pl.BlockSpec((P.shape[1], 128), lambda i, j: (0, j)),
            pl.BlockSpec((1, 128), lambda i, j: (0, j)),
            pl.BlockSpec((bm, 1), lambda i, j: (i, 0)),
            pl.BlockSpec((bm, 1), lambda i, j: (i, 0)),
        ],
        out_specs=pl.BlockSpec((1, bm, 128), lambda i, j: (j, i, 0)),
        out_shape=jax.ShapeDtypeStruct((nc, R, 128), f32),
    )(P, W1, b1.reshape(1, hd), ni.reshape(R, 1), no.reshape(R, 1))


def _m2_body(x_ref, w2_ref, b2_ref, xw_ref, xb2_ref):
    xw_ref[:, :] = lax.dot_general(
        x_ref[:, :], w2_ref[:, :], (((1,), (1,)), ((), ())),
        preferred_element_type=f32)
    xb2_ref[:, :] = jnp.dot(x_ref[:, :], b2_ref[:, :],
                            preferred_element_type=f32)


def _m2_call(X, W2, b2):
    bm = 1024
    bt, od = X.shape
    hd = W2.shape[0]
    return pl.pallas_call(
        _m2_body,
        grid=(bt // bm,),
        in_specs=[
            pl.BlockSpec((bm, od), lambda i: (i, 0)),
            pl.BlockSpec((hd, od), lambda i: (0, 0)),
            pl.BlockSpec((od, 1), lambda i: (0, 0)),
        ],
        out_specs=[
            pl.BlockSpec((bm, hd), lambda i: (i, 0)),
            pl.BlockSpec((bm, 1), lambda i: (i, 0)),
        ],
        out_shape=[
            jax.ShapeDtypeStruct((bt, hd), f32),
            jax.ShapeDtypeStruct((bt, 1), f32),
        ],
    )(X, W2, b2.reshape(od, 1))


def _m3_body(xw_ref, q_ref, ni_ref, xb2_ref, out_ref):
    s = lax.dot_general(xw_ref[:, :], q_ref[:, :], (((1,), (1,)), ((), ())),
                        preferred_element_type=f32)
    out_ref[:, :] = jax.nn.sigmoid(s * ni_ref[:, :] + xb2_ref[:, :])


def _m3_call(XW, Qp, ni_row, xb2):
    bm, bn = 1024, 1280
    bt, hd = XW.shape
    return pl.pallas_call(
        _m3_body,
        grid=(bt // bm, R // bn),
        in_specs=[
            pl.BlockSpec((bm, hd), lambda i, j: (i, 0)),
            pl.BlockSpec((bn, hd), lambda i, j: (j, 0)),
            pl.BlockSpec((1, bn), lambda i, j: (0, j)),
            pl.BlockSpec((bm, 1), lambda i, j: (i, 0)),
        ],
        out_specs=pl.BlockSpec((bm, bn), lambda i, j: (i, j)),
        out_shape=jax.ShapeDtypeStruct((bt, N_NODES), f32),
    )(XW, Qp, ni_row, xb2)


def kernel(X, label_embeds, edge_index, W1, b1, W2, b2):
    ei = edge_index.astype(i32)
    epr = N_EDGES // NSUB
    pad = jnp.full((NSUB, EPS - epr), JUNK, i32)
    src3 = jnp.concatenate([ei[0].reshape(NSUB, epr), pad], 1).reshape(NSUB, NB, BE)
    dst3 = jnp.concatenate([ei[1].reshape(NSUB, epr), pad], 1).reshape(NSUB, NB, BE)
    sd3 = jnp.stack([src3, dst3])
    src3h = src3.reshape(NSUB, 2, NB2, BE)
    dst3h = dst3.reshape(NSUB, 2, NB2, BE)

    deg = _deg_call(sd3)
    norm_out = lax.rsqrt(jnp.maximum(deg[0, :, 0], 1.0))
    norm_in = lax.rsqrt(jnp.maximum(deg[1, :, 0], 1.0))

    in_dim = label_embeds.shape[1]
    F0 = jnp.concatenate(
        [label_embeds, jnp.zeros((R - N_NODES, in_dim), f32)], 0
    ) * norm_out[:, None]
    F0s = F0.reshape(R, 2, in_dim // 2).transpose(1, 0, 2)
    P = _scatter_call(F0s, src3h, dst3h)
    P = P.transpose(1, 0, 2).reshape(R, in_dim)

    hd = W1.shape[1]
    F1s = _m1_call(P, W1, b1, norm_in, norm_out)
    Q = _scatter_call(F1s, src3h, dst3h)
    Q = jnp.concatenate([Q[:8], (Q[8] + Q[9])[None]], 0)
    Qp = Q.transpose(1, 0, 2).reshape(R, hd)

    XW, xb2 = _m2_call(X, W2, b2)
    return _m3_call(XW, Qp, norm_in.reshape(1, R), xb2)


# speedup vs baseline: 1.0943x; 1.0586x over previous
import functools

import jax
import jax.numpy as jnp
from jax import lax
from jax.experimental import pallas as pl
from jax.experimental.pallas import tpu as pltpu
from jax.experimental.pallas import tpu_sc as plsc

f32 = jnp.float32
i32 = jnp.int32

N_NODES = 10000
R = 10240
JUNK = N_NODES
N_EDGES = 160000
NSUB = 16
EPS = R
BE = 128
NB = EPS // BE
STRIPE = R // NSUB

_mesh = plsc.VectorSubcoreMesh(core_axis_name="c", subcore_axis_name="s")


def _fill(ref, val, nrow, ncol):
    def row(r, carry):
        for j in range(ncol // 16):
            ref[r, pl.ds(j * 16, 16)] = jnp.full((16,), val, f32)
        return carry
    lax.fori_loop(0, nrow, row, 0)


def _deg_body(sd3, deg_hbm, idxv, onesv, table):
    cid = lax.axis_index("c")
    sid = lax.axis_index("s")
    base = sid * STRIPE
    _fill(onesv, 0.0, BE, 128)
    for k in range(STRIPE // BE):
        pltpu.sync_copy(onesv, table.at[pl.ds(base + k * BE, BE)])
    _fill(onesv, 1.0, BE, 128)
    pltpu.sync_copy(sd3.at[cid, sid], idxv)
    plsc.subcore_barrier()

    def batch(b, carry):
        pltpu.sync_copy(onesv, table.at[idxv.at[b]], add=True)
        return carry
    lax.fori_loop(0, NB, batch, 0)
    plsc.subcore_barrier()
    pltpu.sync_copy(table.at[pl.ds(base, STRIPE)],
                    deg_hbm.at[cid, pl.ds(base, STRIPE)])


def _deg_call(sd3):
    return pl.kernel(
        _deg_body,
        out_type=jax.ShapeDtypeStruct((2, R, 128), f32),
        mesh=_mesh,
        scratch_types=[
            pltpu.VMEM((NB, BE), i32),
            pltpu.VMEM((BE, 128), f32),
            pltpu.VMEM_SHARED((R, 128), f32),
        ],
    )(sd3)


NB2 = NB // 2


def _scatter_body(nchunk, dc, fstack, src3h, dst3h, out, sidxv, didxv, rowsa,
                  rowsb, table, sema, semb):
    cid = lax.axis_index("c")
    sid = lax.axis_index("s")
    base = sid * STRIPE
    c0 = nchunk // 2

    def chunk_body(chunk, oslot, halves):
        _fill(rowsa, 0.0, BE, dc)
        for k in range(STRIPE // BE):
            pltpu.sync_copy(rowsa, table.at[pl.ds(base + k * BE, BE)])
        plsc.subcore_barrier()
        src = fstack.at[chunk]

        def gfire(b, buf, sem):
            pltpu.async_copy(src.at[sidxv.at[b, pl.ds(0, 64)]],
                             buf.at[pl.ds(0, 64)], sem)
            pltpu.async_copy(src.at[sidxv.at[b, pl.ds(64, 64)]],
                             buf.at[pl.ds(64, 64)], sem)

        def gwait(buf, sem):
            pltpu.make_async_copy(src.at[sidxv.at[0]], buf, sem).wait()

        for h in halves:
            pltpu.sync_copy(src3h.at[sid, h], sidxv)
            pltpu.sync_copy(dst3h.at[sid, h], didxv)
            gfire(0, rowsa, sema)
            gfire(1, rowsb, semb)

            def pair(p, carry):
                b0 = 2 * p
                gwait(rowsa, sema)
                pltpu.sync_copy(rowsa, table.at[didxv.at[b0]], add=True)

                @pl.when(p < NB2 // 2 - 1)
                def _():
                    gfire(b0 + 2, rowsa, sema)
                gwait(rowsb, semb)
                pltpu.sync_copy(rowsb, table.at[didxv.at[b0 + 1]], add=True)

                @pl.when(p < NB2 // 2 - 1)
                def _():
                    gfire(b0 + 3, rowsb, semb)
                return carry
            lax.fori_loop(0, NB2 // 2, pair, 0)
        plsc.subcore_barrier()
        pltpu.sync_copy(table.at[pl.ds(base, STRIPE)],
                        out.at[oslot, pl.ds(base, STRIPE)])

    for i in range(c0):
        chunk_body(cid * c0 + i, cid * c0 + i, (0, 1))
    if nchunk % 2:
        chunk_body(nchunk - 1, nchunk - 1 + cid, (cid,))


def _scatter_call(fstack, src3h, dst3h):
    nchunk, _, dc = fstack.shape
    body = functools.partial(_scatter_body, nchunk, dc)
    return pl.kernel(
        body,
        out_type=jax.ShapeDtypeStruct((nchunk + nchunk % 2, R, dc), f32),
        mesh=_mesh,
        scratch_types=[
            pltpu.VMEM((NB2, BE), i32),
            pltpu.VMEM((NB2, BE), i32),
            pltpu.VMEM((BE, dc), f32),
            pltpu.VMEM((BE, dc), f32),
            pltpu.VMEM_SHARED((R, dc), f32),
            pltpu.SemaphoreType.DMA,
            pltpu.SemaphoreType.DMA,
        ],
    )(fstack, src3h, dst3h)


def _m1_body(p_ref, w1_ref, b1_ref, ni_ref, no_ref, out_ref):
    acc = jnp.dot(p_ref[:, :], w1_ref[:, :], preferred_element_type=f32)
    out_ref[0, :, :] = (jnp.maximum(acc * ni_ref[:, :] + b1_ref[:, :], 0.0)
                        * no_ref[:, :])


def _m1_call(P, W1, b1, ni, no):
    bm = 1024
    hd = W1.shape[1]
    nc = hd // 128
    return pl.pallas_call(
        _m1_body,
        grid=(R // bm, nc),
        in_specs=[
            pl.BlockSpec((bm, P.shape[1]), lambda i, j: (i, 0)),
            pl.BlockSpec((P.shape[1], 128), lambda i, j: (0, j)),
            pl.BlockSpec((1, 128), lambda i, j: (0, j)),
            pl.BlockSpec((bm, 1), lambda i, j: (i, 0)),
            pl.BlockSpec((bm, 1), lambda i, j: (i, 0)),
        ],
        out_specs=pl.BlockSpec((1, bm, 128), lambda i, j: (j, i, 0)),
        out_shape=jax.ShapeDtypeStruct((nc, R, 128), f32),
    )(P, W1, b1.reshape(1, hd), ni.reshape(R, 1), no.reshape(R, 1))


def _m2_body(x_ref, w2_ref, b2_ref, xw_ref, xb2_ref):
    xw_ref[:, :] = lax.dot_general(
        x_ref[:, :], w2_ref[:, :], (((1,), (1,)), ((), ())),
        preferred_element_type=f32)
    xb2_ref[:, :] = jnp.dot(x_ref[:, :], b2_ref[:, :],
                            preferred_element_type=f32)


def _m2_call(X, W2, b2):
    bm = 1024
    bt, od = X.shape
    hd = W2.shape[0]
    return pl.pallas_call(
        _m2_body,
        grid=(bt // bm,),
        in_specs=[
            pl.BlockSpec((bm, od), lambda i: (i, 0)),
            pl.BlockSpec((hd, od), lambda i: (0, 0)),
            pl.BlockSpec((od, 1), lambda i: (0, 0)),
        ],
        out_specs=[
            pl.BlockSpec((bm, hd), lambda i: (i, 0)),
            pl.BlockSpec((bm, 1), lambda i: (i, 0)),
        ],
        out_shape=[
            jax.ShapeDtypeStruct((bt, hd), f32),
            jax.ShapeDtypeStruct((bt, 1), f32),
        ],
    )(X, W2, b2.reshape(od, 1))


def _m3_body(xw_ref, q_ref, ni_ref, xb2_ref, out_ref):
    s = lax.dot_general(xw_ref[:, :], q_ref[:, :], (((1,), (1,)), ((), ())),
                        preferred_element_type=f32)
    out_ref[:, :] = jax.nn.sigmoid(s * ni_ref[:, :] + xb2_ref[:, :])


def _m3_call(XW, Qp, ni_row, xb2):
    bm, bn = 1024, 1280
    bt, hd = XW.shape
    return pl.pallas_call(
        _m3_body,
        grid=(bt // bm, R // bn),
        in_specs=[
            pl.BlockSpec((bm, hd), lambda i, j: (i, 0)),
            pl.BlockSpec((bn, hd), lambda i, j: (j, 0)),
            pl.BlockSpec((1, bn), lambda i, j: (0, j)),
            pl.BlockSpec((bm, 1), lambda i, j: (i, 0)),
        ],
        out_specs=pl.BlockSpec((bm, bn), lambda i, j: (i, j)),
        out_shape=jax.ShapeDtypeStruct((bt, N_NODES), f32),
    )(XW, Qp, ni_row, xb2)


def kernel(X, label_embeds, edge_index, W1, b1, W2, b2):
    ei = edge_index.astype(i32)
    epr = N_EDGES // NSUB
    pad = jnp.full((NSUB, EPS - epr), JUNK, i32)
    src3 = jnp.concatenate([ei[0].reshape(NSUB, epr), pad], 1).reshape(NSUB, NB, BE)
    dst3 = jnp.concatenate([ei[1].reshape(NSUB, epr), pad], 1).reshape(NSUB, NB, BE)
    sd3 = jnp.stack([src3, dst3])
    src3h = src3.reshape(NSUB, 2, NB2, BE)
    dst3h = dst3.reshape(NSUB, 2, NB2, BE)

    deg = _deg_call(sd3)
    norm_out = lax.rsqrt(jnp.maximum(deg[0, :, 0], 1.0))
    norm_in = lax.rsqrt(jnp.maximum(deg[1, :, 0], 1.0))

    in_dim = label_embeds.shape[1]
    F0 = jnp.concatenate(
        [label_embeds, jnp.zeros((R - N_NODES, in_dim), f32)], 0
    ) * norm_out[:, None]
    F0s = F0.reshape(R, 2, in_dim // 2).transpose(1, 0, 2)
    P = _scatter_call(F0s, src3h, dst3h)
    P = P.transpose(1, 0, 2).reshape(R, in_dim)

    hd = W1.shape[1]
    F1s = _m1_call(P, W1, b1, norm_in, norm_out)
    Q = _scatter_call(F1s, src3h, dst3h)
    Q = jnp.concatenate([Q[:8], (Q[8] + Q[9])[None]], 0)
    Qp = Q.transpose(1, 0, 2).reshape(R, hd)

    XW, xb2 = _m2_call(X, W2, b2)
    return _m3_call(XW, Qp, norm_in.reshape(1, R), xb2)


# speedup vs baseline: 1.0944x; 1.0002x over previous
import functools

import jax
import jax.numpy as jnp
from jax import lax
from jax.experimental import pallas as pl
from jax.experimental.pallas import tpu as pltpu
from jax.experimental.pallas import tpu_sc as plsc

f32 = jnp.float32
i32 = jnp.int32

N_NODES = 10000
R = 10240
JUNK = N_NODES
N_EDGES = 160000
NSUB = 16
EPS = R
BE = 128
NB = EPS // BE
STRIPE = R // NSUB

_mesh = plsc.VectorSubcoreMesh(core_axis_name="c", subcore_axis_name="s")


def _fill(ref, val, nrow, ncol):
    def row(r, carry):
        for j in range(ncol // 16):
            ref[r, pl.ds(j * 16, 16)] = jnp.full((16,), val, f32)
        return carry
    lax.fori_loop(0, nrow, row, 0)


def _deg_body(sd3, deg_hbm, idxv, onesv, table):
    cid = lax.axis_index("c")
    sid = lax.axis_index("s")
    base = sid * STRIPE
    _fill(onesv, 0.0, BE, 128)
    for k in range(STRIPE // BE):
        pltpu.sync_copy(onesv, table.at[pl.ds(base + k * BE, BE)])
    _fill(onesv, 1.0, BE, 128)
    pltpu.sync_copy(sd3.at[cid, sid], idxv)
    plsc.subcore_barrier()

    def batch(b, carry):
        pltpu.sync_copy(onesv, table.at[idxv.at[b]], add=True)
        return carry
    lax.fori_loop(0, NB, batch, 0)
    plsc.subcore_barrier()
    pltpu.sync_copy(table.at[pl.ds(base, STRIPE)],
                    deg_hbm.at[cid, pl.ds(base, STRIPE)])


def _deg_call(sd3):
    return pl.kernel(
        _deg_body,
        out_type=jax.ShapeDtypeStruct((2, R, 128), f32),
        mesh=_mesh,
        scratch_types=[
            pltpu.VMEM((NB, BE), i32),
            pltpu.VMEM((BE, 128), f32),
            pltpu.VMEM_SHARED((R, 128), f32),
        ],
    )(sd3)


NB2 = NB // 2


def _scatter_body(nchunk, dc, fstack, src3h, dst3h, out, sidxv, didxv, rowsa,
                  rowsb, table, sema, semb):
    cid = lax.axis_index("c")
    sid = lax.axis_index("s")
    base = sid * STRIPE
    c0 = nchunk // 2

    def chunk_body(chunk, oslot, halves):
        _fill(rowsa, 0.0, BE, dc)
        for k in range(STRIPE // BE):
            pltpu.sync_copy(rowsa, table.at[pl.ds(base + k * BE, BE)])
        plsc.subcore_barrier()
        src = fstack.at[chunk]

        def gfire(b, buf, sem):
            for q in range(4):
                pltpu.async_copy(src.at[sidxv.at[b, pl.ds(32 * q, 32)]],
                                 buf.at[pl.ds(32 * q, 32)], sem)

        def gwait(buf, sem):
            pltpu.make_async_copy(src.at[sidxv.at[0]], buf, sem).wait()

        for h in halves:
            pltpu.sync_copy(src3h.at[sid, h], sidxv)
            pltpu.sync_copy(dst3h.at[sid, h], didxv)
            gfire(0, rowsa, sema)
            gfire(1, rowsb, semb)

            def pair(p, carry):
                b0 = 2 * p
                gwait(rowsa, sema)
                pltpu.sync_copy(rowsa, table.at[didxv.at[b0]], add=True)

                @pl.when(p < NB2 // 2 - 1)
                def _():
                    gfire(b0 + 2, rowsa, sema)
                gwait(rowsb, semb)
                pltpu.sync_copy(rowsb, table.at[didxv.at[b0 + 1]], add=True)

                @pl.when(p < NB2 // 2 - 1)
                def _():
                    gfire(b0 + 3, rowsb, semb)
                return carry
            lax.fori_loop(0, NB2 // 2, pair, 0)
        plsc.subcore_barrier()
        pltpu.sync_copy(table.at[pl.ds(base, STRIPE)],
                        out.at[oslot, pl.ds(base, STRIPE)])

    for i in range(c0):
        chunk_body(cid * c0 + i, cid * c0 + i, (0, 1))
    if nchunk % 2:
        chunk_body(nchunk - 1, nchunk - 1 + cid, (cid,))


def _scatter_call(fstack, src3h, dst3h):
    nchunk, _, dc = fstack.shape
    body = functools.partial(_scatter_body, nchunk, dc)
    return pl.kernel(
        body,
        out_type=jax.ShapeDtypeStruct((nchunk + nchunk % 2, R, dc), f32),
        mesh=_mesh,
        scratch_types=[
            pltpu.VMEM((NB2, BE), i32),
            pltpu.VMEM((NB2, BE), i32),
            pltpu.VMEM((BE, dc), f32),
            pltpu.VMEM((BE, dc), f32),
            pltpu.VMEM_SHARED((R, dc), f32),
            pltpu.SemaphoreType.DMA,
            pltpu.SemaphoreType.DMA,
        ],
    )(fstack, src3h, dst3h)


def _m1_body(p_ref, w1_ref, b1_ref, ni_ref, no_ref, out_ref):
    acc = jnp.dot(p_ref[:, :], w1_ref[:, :], preferred_element_type=f32)
    out_ref[0, :, :] = (jnp.maximum(acc * ni_ref[:, :] + b1_ref[:, :], 0.0)
                        * no_ref[:, :])


def _m1_call(P, W1, b1, ni, no):
    bm = 1024
    hd = W1.shape[1]
    nc = hd // 128
    return pl.pallas_call(
        _m1_body,
        grid=(R // bm, nc),
        in_specs=[
            pl.BlockSpec((bm, P.shape[1]), lambda i, j: (i, 0)),
            pl.BlockSpec((P.shape[1], 128), lambda i, j: (0, j)),
            pl.BlockSpec((1, 128), lambda i, j: (0, j)),
            pl.BlockSpec((bm, 1), lambda i, j: (i, 0)),
            pl.BlockSpec((bm, 1), lambda i, j: (i, 0)),
        ],
        out_specs=pl.BlockSpec((1, bm, 128), lambda i, j: (j, i, 0)),
        out_shape=jax.ShapeDtypeStruct((nc, R, 128), f32),
    )(P, W1, b1.reshape(1, hd), ni.reshape(R, 1), no.reshape(R, 1))


def _m2_body(x_ref, w2_ref, b2_ref, xw_ref, xb2_ref):
    xw_ref[:, :] = lax.dot_general(
        x_ref[:, :], w2_ref[:, :], (((1,), (1,)), ((), ())),
        preferred_element_type=f32)
    xb2_ref[:, :] = jnp.dot(x_ref[:, :], b2_ref[:, :],
                            preferred_element_type=f32)


def _m2_call(X, W2, b2):
    bm = 1024
    bt, od = X.shape
    hd = W2.shape[0]
    return pl.pallas_call(
        _m2_body,
        grid=(bt // bm,),
        in_specs=[
            pl.BlockSpec((bm, od), lambda i: (i, 0)),
            pl.BlockSpec((hd, od), lambda i: (0, 0)),
            pl.BlockSpec((od, 1), lambda i: (0, 0)),
        ],
        out_specs=[
            pl.BlockSpec((bm, hd), lambda i: (i, 0)),
            pl.BlockSpec((bm, 1), lambda i: (i, 0)),
        ],
        out_shape=[
            jax.ShapeDtypeStruct((bt, hd), f32),
            jax.ShapeDtypeStruct((bt, 1), f32),
        ],
    )(X, W2, b2.reshape(od, 1))


def _m3_body(xw_ref, q_ref, ni_ref, xb2_ref, out_ref):
    s = lax.dot_general(xw_ref[:, :], q_ref[:, :], (((1,), (1,)), ((), ())),
                        preferred_element_type=f32)
    out_ref[:, :] = jax.nn.sigmoid(s * ni_ref[:, :] + xb2_ref[:, :])


def _m3_call(XW, Qp, ni_row, xb2):
    bm, bn = 1024, 1280
    bt, hd = XW.shape
    return pl.pallas_call(
        _m3_body,
        grid=(bt // bm, R // bn),
        in_specs=[
            pl.BlockSpec((bm, hd), lambda i, j: (i, 0)),
            pl.BlockSpec((bn, hd), lambda i, j: (j, 0)),
            pl.BlockSpec((1, bn), lambda i, j: (0, j)),
            pl.BlockSpec((bm, 1), lambda i, j: (i, 0)),
        ],
        out_specs=pl.BlockSpec((bm, bn), lambda i, j: (i, j)),
        out_shape=jax.ShapeDtypeStruct((bt, N_NODES), f32),
    )(XW, Qp, ni_row, xb2)


def kernel(X, label_embeds, edge_index, W1, b1, W2, b2):
    ei = edge_index.astype(i32)
    epr = N_EDGES // NSUB
    pad = jnp.full((NSUB, EPS - epr), JUNK, i32)
    src3 = jnp.concatenate([ei[0].reshape(NSUB, epr), pad], 1).reshape(NSUB, NB, BE)
    dst3 = jnp.concatenate([ei[1].reshape(NSUB, epr), pad], 1).reshape(NSUB, NB, BE)
    sd3 = jnp.stack([src3, dst3])
    src3h = src3.reshape(NSUB, 2, NB2, BE)
    dst3h = dst3.reshape(NSUB, 2, NB2, BE)

    deg = _deg_call(sd3)
    norm_out = lax.rsqrt(jnp.maximum(deg[0, :, 0], 1.0))
    norm_in = lax.rsqrt(jnp.maximum(deg[1, :, 0], 1.0))

    in_dim = label_embeds.shape[1]
    F0 = jnp.concatenate(
        [label_embeds, jnp.zeros((R - N_NODES, in_dim), f32)], 0
    ) * norm_out[:, None]
    F0s = F0.reshape(R, 2, in_dim // 2).transpose(1, 0, 2)
    P = _scatter_call(F0s, src3h, dst3h)
    P = P.transpose(1, 0, 2).reshape(R, in_dim)

    hd = W1.shape[1]
    F1s = _m1_call(P, W1, b1, norm_in, norm_out)
    Q = _scatter_call(F1s, src3h, dst3h)
    Q = jnp.concatenate([Q[:8], (Q[8] + Q[9])[None]], 0)
    Qp = Q.transpose(1, 0, 2).reshape(R, hd)

    XW, xb2 = _m2_call(X, W2, b2)
    return _m3_call(XW, Qp, norm_in.reshape(1, R), xb2)


# speedup vs baseline: 1.0993x; 1.0044x over previous
import functools

import jax
import jax.numpy as jnp
from jax import lax
from jax.experimental import pallas as pl
from jax.experimental.pallas import tpu as pltpu
from jax.experimental.pallas import tpu_sc as plsc

f32 = jnp.float32
i32 = jnp.int32

N_NODES = 10000
R = 10240
JUNK = N_NODES
N_EDGES = 160000
NSUB = 16
EPS = R
BE = 128
NB = EPS // BE
STRIPE = R // NSUB

_mesh = plsc.VectorSubcoreMesh(core_axis_name="c", subcore_axis_name="s")


def _fill(ref, val, nrow, ncol):
    def row(r, carry):
        for j in range(ncol // 16):
            ref[r, pl.ds(j * 16, 16)] = jnp.full((16,), val, f32)
        return carry
    lax.fori_loop(0, nrow, row, 0)


def _deg_body(sd3, deg_hbm, idxv, onesv, table):
    cid = lax.axis_index("c")
    sid = lax.axis_index("s")
    base = sid * STRIPE
    _fill(onesv, 0.0, BE, 128)
    for k in range(STRIPE // BE):
        pltpu.sync_copy(onesv, table.at[pl.ds(base + k * BE, BE)])
    _fill(onesv, 1.0, BE, 128)
    pltpu.sync_copy(sd3.at[cid, sid], idxv)
    plsc.subcore_barrier()

    def batch(b, carry):
        pltpu.sync_copy(onesv, table.at[idxv.at[b]], add=True)
        return carry
    lax.fori_loop(0, NB, batch, 0)
    plsc.subcore_barrier()
    pltpu.sync_copy(table.at[pl.ds(base, STRIPE)],
                    deg_hbm.at[cid, pl.ds(base, STRIPE)])


def _deg_call(sd3):
    return pl.kernel(
        _deg_body,
        out_type=jax.ShapeDtypeStruct((2, R, 128), f32),
        mesh=_mesh,
        scratch_types=[
            pltpu.VMEM((NB, BE), i32),
            pltpu.VMEM((BE, 128), f32),
            pltpu.VMEM_SHARED((R, 128), f32),
        ],
    )(sd3)


NB2 = NB // 2


def _scatter_body(nchunk, dc, fstack, src3h, dst3h, out, sidxv, didxv, rowsa,
                  rowsb, table, sema, semb):
    cid = lax.axis_index("c")
    sid = lax.axis_index("s")
    base = sid * STRIPE
    c0 = nchunk // 2

    def chunk_body(chunk, oslot, halves):
        _fill(rowsa, 0.0, BE, dc)
        for k in range(STRIPE // BE):
            pltpu.sync_copy(rowsa, table.at[pl.ds(base + k * BE, BE)])
        plsc.subcore_barrier()
        src = fstack.at[chunk]

        def gfire(b, buf, sem):
            for q in range(4):
                pltpu.async_copy(src.at[sidxv.at[b, pl.ds(32 * q, 32)]],
                                 buf.at[pl.ds(32 * q, 32)], sem)

        def gwait(buf, sem):
            pltpu.make_async_copy(src.at[sidxv.at[0]], buf, sem).wait()

        for h in halves:
            pltpu.sync_copy(src3h.at[sid, h], sidxv)
            pltpu.sync_copy(dst3h.at[sid, h], didxv)
            gfire(0, rowsa, sema)
            gfire(1, rowsb, semb)

            def pair(p, carry):
                b0 = 2 * p
                gwait(rowsa, sema)
                pltpu.sync_copy(rowsa, table.at[didxv.at[b0]], add=True)

                @pl.when(p < NB2 // 2 - 1)
                def _():
                    gfire(b0 + 2, rowsa, sema)
                gwait(rowsb, semb)
                pltpu.sync_copy(rowsb, table.at[didxv.at[b0 + 1]], add=True)

                @pl.when(p < NB2 // 2 - 1)
                def _():
                    gfire(b0 + 3, rowsb, semb)
                return carry
            lax.fori_loop(0, NB2 // 2, pair, 0)
        plsc.subcore_barrier()
        pltpu.sync_copy(table.at[pl.ds(base, STRIPE)],
                        out.at[oslot, pl.ds(base, STRIPE)])

    for i in range(c0):
        chunk_body(cid * c0 + i, cid * c0 + i, (0, 1))
    if nchunk % 2:
        chunk_body(nchunk - 1, nchunk - 1 + cid, (cid,))


def _scatter_call(fstack, src3h, dst3h):
    nchunk, _, dc = fstack.shape
    body = functools.partial(_scatter_body, nchunk, dc)
    return pl.kernel(
        body,
        out_type=jax.ShapeDtypeStruct((nchunk + nchunk % 2, R, dc), f32),
        mesh=_mesh,
        scratch_types=[
            pltpu.VMEM((NB2, BE), i32),
            pltpu.VMEM((NB2, BE), i32),
            pltpu.VMEM((BE, dc), f32),
            pltpu.VMEM((BE, dc), f32),
            pltpu.VMEM_SHARED((R, dc), f32),
            pltpu.SemaphoreType.DMA,
            pltpu.SemaphoreType.DMA,
        ],
    )(fstack, src3h, dst3h)


def _f0_body(emb_ref, no_ref, out_ref):
    out_ref[0, :, :] = emb_ref[:, :] * no_ref[:, :]


def _f0_call(emb, no):
    bm = 1024
    return pl.pallas_call(
        _f0_body,
        grid=(R // bm, 2),
        in_specs=[
            pl.BlockSpec((bm, 128), lambda i, j: (i, j)),
            pl.BlockSpec((bm, 1), lambda i, j: (i, 0)),
        ],
        out_specs=pl.BlockSpec((1, bm, 128), lambda i, j: (j, i, 0)),
        out_shape=jax.ShapeDtypeStruct((2, R, 128), f32),
    )(emb, no.reshape(R, 1))


def _m1_body(p_ref, w1_ref, b1_ref, ni_ref, no_ref, out_ref):
    acc = jnp.dot(p_ref[:, :], w1_ref[:, :], preferred_element_type=f32)
    out_ref[0, :, :] = (jnp.maximum(acc * ni_ref[:, :] + b1_ref[:, :], 0.0)
                        * no_ref[:, :])


def _m1_call(P, W1, b1, ni, no):
    bm = 1024
    hd = W1.shape[1]
    nc = hd // 128
    return pl.pallas_call(
        _m1_body,
        grid=(R // bm, nc),
        in_specs=[
            pl.BlockSpec((bm, P.shape[1]), lambda i, j: (i, 0)),
            pl.BlockSpec((P.shape[1], 128), lambda i, j: (0, j)),
            pl.BlockSpec((1, 128), lambda i, j: (0, j)),
            pl.BlockSpec((bm, 1), lambda i, j: (i, 0)),
            pl.BlockSpec((bm, 1), lambda i, j: (i, 0)),
        ],
        out_specs=pl.BlockSpec((1, bm, 128), lambda i, j: (j, i, 0)),
        out_shape=jax.ShapeDtypeStruct((nc, R, 128), f32),
    )(P, W1, b1.reshape(1, hd), ni.reshape(R, 1), no.reshape(R, 1))


def _m2_body(x_ref, w2_ref, b2_ref, xw_ref, xb2_ref):
    xw_ref[:, :] = lax.dot_general(
        x_ref[:, :], w2_ref[:, :], (((1,), (1,)), ((), ())),
        preferred_element_type=f32)
    xb2_ref[:, :] = jnp.dot(x_ref[:, :], b2_ref[:, :],
                            preferred_element_type=f32)


def _m2_call(X, W2, b2):
    bm = 1024
    bt, od = X.shape
    hd = W2.shape[0]
    return pl.pallas_call(
        _m2_body,
        grid=(bt // bm,),
        in_specs=[
            pl.BlockSpec((bm, od), lambda i: (i, 0)),
            pl.BlockSpec((hd, od), lambda i: (0, 0)),
            pl.BlockSpec((od, 1), lambda i: (0, 0)),
        ],
        out_specs=[
            pl.BlockSpec((bm, hd), lambda i: (i, 0)),
            pl.BlockSpec((bm, 1), lambda i: (i, 0)),
        ],
        out_shape=[
            jax.ShapeDtypeStruct((bt, hd), f32),
            jax.ShapeDtypeStruct((bt, 1), f32),
        ],
    )(X, W2, b2.reshape(od, 1))


def _m3_body(xw_ref, q_ref, ni_ref, xb2_ref, out_ref):
    s = lax.dot_general(xw_ref[:, :], q_ref[:, :], (((1,), (1,)), ((), ())),
                        preferred_element_type=f32)
    out_ref[:, :] = jax.nn.sigmoid(s * ni_ref[:, :] + xb2_ref[:, :])


def _m3_call(XW, Qp, ni_row, xb2):
    bm, bn = 1024, 1280
    bt, hd = XW.shape
    return pl.pallas_call(
        _m3_body,
        grid=(bt // bm, R // bn),
        in_specs=[
            pl.BlockSpec((bm, hd), lambda i, j: (i, 0)),
            pl.BlockSpec((bn, hd), lambda i, j: (j, 0)),
            pl.BlockSpec((1, bn), lambda i, j: (0, j)),
            pl.BlockSpec((bm, 1), lambda i, j: (i, 0)),
        ],
        out_specs=pl.BlockSpec((bm, bn), lambda i, j: (i, j)),
        out_shape=jax.ShapeDtypeStruct((bt, N_NODES), f32),
    )(XW, Qp, ni_row, xb2)


def kernel(X, label_embeds, edge_index, W1, b1, W2, b2):
    ei = edge_index.astype(i32)
    epr = N_EDGES // NSUB
    pad = jnp.full((NSUB, EPS - epr), JUNK, i32)
    src3 = jnp.concatenate([ei[0].reshape(NSUB, epr), pad], 1).reshape(NSUB, NB, BE)
    dst3 = jnp.concatenate([ei[1].reshape(NSUB, epr), pad], 1).reshape(NSUB, NB, BE)
    sd3 = jnp.stack([src3, dst3])
    src3h = src3.reshape(NSUB, 2, NB2, BE)
    dst3h = dst3.reshape(NSUB, 2, NB2, BE)

    deg = _deg_call(sd3)
    norm_out = lax.rsqrt(jnp.maximum(deg[0, :, 0], 1.0))
    norm_in = lax.rsqrt(jnp.maximum(deg[1, :, 0], 1.0))

    in_dim = label_embeds.shape[1]
    F0s = _f0_call(label_embeds, norm_out)
    P = _scatter_call(F0s, src3h, dst3h)
    P = P.transpose(1, 0, 2).reshape(R, in_dim)

    hd = W1.shape[1]
    F1s = _m1_call(P, W1, b1, norm_in, norm_out)
    Q = _scatter_call(F1s, src3h, dst3h)
    Q = jnp.concatenate([Q[:8], (Q[8] + Q[9])[None]], 0)
    Qp = Q.transpose(1, 0, 2).reshape(R, hd)

    XW, xb2 = _m2_call(X, W2, b2)
    return _m3_call(XW, Qp, norm_in.reshape(1, R), xb2)
